# baseline (speedup 1.0000x reference)
import jax
import jax.numpy as jnp
from jax import lax
from jax.experimental import pallas as pl
from jax.experimental.pallas import tpu as pltpu
from jax.experimental.pallas import tpu_sc as plsc

NC, NS, LANES = 2, 16, 16
NW = NC * NS
B, L, H = 4096, 50, 32
UPW = B // NW
IPW = UPW * L
NCHUNK = IPW // LANES


def _mf_body(user_hbm, item_hbm, target_hbm, uw_hbm, iw_hbm, ub_hbm, ib_hbm,
             bias_hbm, pred_hbm, loss_hbm,
             uidx_v, iidx_v, urows_v, ub_v, rows_v, ibv_v, tgt_v, pred_v,
             bias_v, acc_v, sem_u, sem_r, sem_b):
    wid = lax.axis_index("s") * NC + lax.axis_index("c")

    pltpu.sync_copy(user_hbm.at[wid], uidx_v)
    pltpu.sync_copy(item_hbm.at[wid], iidx_v)
    pltpu.sync_copy(target_hbm.at[wid], tgt_v)
    pltpu.sync_copy(bias_hbm, bias_v)
    pltpu.async_copy(uw_hbm.at[uidx_v], urows_v, sem_u).wait()
    pltpu.async_copy(ub_hbm.at[uidx_v], ub_v, sem_u).wait()

    iota = lax.iota(jnp.int32, LANES)
    zeros16 = jnp.zeros((LANES,), jnp.int32)
    nchunk_u = (L + LANES - 1) // LANES
    l_idx = [jnp.minimum(c * LANES + iota, L - 1) for c in range(nchunk_u)]

    def user_body(u, carry):
        cr = pltpu.async_copy(iw_hbm.at[iidx_v.at[u]], rows_v, sem_r)
        cb = pltpu.async_copy(ib_hbm.at[iidx_v.at[u]], ibv_v, sem_b)
        cr.wait()
        cb.wait()
        u16 = jnp.full((LANES,), u, jnp.int32)
        ubv = plsc.load_gather(ub_v, [u16])
        accs = [jnp.zeros((LANES,), jnp.float32)] * nchunk_u
        su = jnp.zeros((LANES,), jnp.float32)
        for hg in range(0, H, LANES):
            uhs = []
            for hh in range(LANES):
                h16 = jnp.full((LANES,), hg + hh, jnp.int32)
                uh = plsc.load_gather(urows_v, [u16, h16]) + ubv
                uhs.append(uh)
                su = su + uh
            for c in range(nchunk_u):
                for hh in range(LANES):
                    h16 = jnp.full((LANES,), hg + hh, jnp.int32)
                    col = plsc.load_gather(rows_v, [l_idx[c], h16])
                    accs[c] = accs[c] + uhs[hh] * col
        base_vec = u * L + iota
        for c in range(nchunk_u):
            ibv = plsc.load_gather(ibv_v, [l_idx[c]])
            predc = accs[c] + ibv * su
            valid = (c * LANES + iota) < L
            plsc.store_scatter(pred_v, [base_vec + c * LANES], predc,
                               mask=valid)
        return carry

    lax.fori_loop(0, UPW, user_body, 0)

    def loss_body(i, acc):
        off = pl.multiple_of(i * LANES, LANES)
        p = pred_v[pl.ds(off, LANES)] + bias_v[...]
        pred_v[pl.ds(off, LANES)] = p
        e = p - tgt_v[pl.ds(off, LANES)]
        return acc + e * e

    acc = lax.fori_loop(0, NCHUNK, loss_body, jnp.zeros((LANES,), jnp.float32))
    acc_v[...] = acc * (1.0 / (B * L))

    pltpu.sync_copy(pred_v, pred_hbm.at[wid])
    pltpu.sync_copy(acc_v, loss_hbm.at[wid])


@jax.jit
def _mf_sc(user_r, item_r, target_r, uw, iw, ub, ib, bias16):
    mesh = plsc.VectorSubcoreMesh(core_axis_name="c", subcore_axis_name="s",
                                  num_cores=NC, num_subcores=NS)
    f = pl.kernel(
        _mf_body,
        out_type=(jax.ShapeDtypeStruct((NW, IPW), jnp.float32),
                  jax.ShapeDtypeStruct((NW, LANES), jnp.float32)),
        mesh=mesh,
        compiler_params=pltpu.CompilerParams(needs_layout_passes=False,
                                             use_tc_tiling_on_sc=False),
        scratch_types=[
            pltpu.VMEM((UPW,), jnp.int32),
            pltpu.VMEM((UPW, L), jnp.int32),
            pltpu.VMEM((UPW, H), jnp.float32),
            pltpu.VMEM((UPW,), jnp.float32),
            pltpu.VMEM((L, H), jnp.float32),
            pltpu.VMEM((L,), jnp.float32),
            pltpu.VMEM((IPW,), jnp.float32),
            pltpu.VMEM((IPW,), jnp.float32),
            pltpu.VMEM((LANES,), jnp.float32),
            pltpu.VMEM((LANES,), jnp.float32),
            pltpu.SemaphoreType.DMA,
            pltpu.SemaphoreType.DMA,
            pltpu.SemaphoreType.DMA,
        ],
    )
    return f(user_r, item_r, target_r, uw, iw, ub, ib, bias16)


def kernel(user, item, target, user_weight, item_weight, user_bias, item_bias,
           bias):
    user_r = user.reshape(NW, UPW)
    item_r = item.reshape(NW, UPW, L)
    target_r = target.reshape(NW, IPW)
    bias16 = jnp.broadcast_to(bias, (LANES,))
    pred, lossp = _mf_sc(user_r, item_r, target_r, user_weight, item_weight,
                         user_bias.reshape(-1), item_bias.reshape(-1), bias16)
    return pred.reshape(B * L), jnp.sum(lossp)
